# baseline (speedup 1.0000x reference)
import functools

import jax
import jax.numpy as jnp
from jax import lax
from jax.experimental import pallas as pl
from jax.experimental.pallas import tpu as pltpu
from jax.experimental.pallas import tpu_sc as plsc

TRAIN_START = 900000
DIM = 32
LANES = 16

NC = 2
NS = 16
NW = NC * NS

CH = 512
GBLK = 64
NBLK = CH // GBLK
TB = LANES
TSLOT = CH + 2 * LANES
TBH = CH + LANES


def _body(n_chunks, frozen_hbm, trainable_hbm, idx_hbm, out_hbm,
          idx_v, fidx_v, tval_v, tbuf, obuf,
          semf0, semf1, semt0, semt1, semo0, semo1):
    c = lax.axis_index("c")
    s = lax.axis_index("s")
    wid = s * NC + c
    base = wid * (n_chunks * CH)
    lane = lax.iota(jnp.int32, LANES)
    semf = (semf0, semf1)
    semt = (semt0, semt1)
    semo = (semo0, semo1)

    def stage_issue(ci, p):
        pltpu.sync_copy(idx_hbm.at[pl.ds(base + ci * CH, CH)], idx_v)

        nt = jnp.int32(0)
        for g in range(CH // LANES):
            iv = idx_v[pl.ds(g * LANES, LANES)]
            is_t = iv >= TRAIN_START
            dummy = lane + (wid * CH + g * LANES)
            gpr = GBLK // LANES
            fidx_v[p * NBLK + g // gpr, pl.ds((g % gpr) * LANES, LANES)] = (
                jnp.where(is_t, dummy, iv))
            packed = ((iv - TRAIN_START) << 9) | (lane + g * LANES)
            cnt = lax.cumsum(is_t.astype(jnp.int32), axis=0)
            pos = jnp.where(is_t, (p * TSLOT + nt - 1) + cnt,
                            (p * TSLOT + CH + LANES) + lane)
            plsc.store_scatter(tval_v, [pos], packed)
            nt = nt + cnt[LANES - 1]

        for j in range(NBLK):
            pltpu.async_copy(
                frozen_hbm.at[fidx_v.at[p * NBLK + j]],
                obuf.at[pl.ds(p * CH + j * GBLK, GBLK)], semf[p])

        last = jnp.full((LANES,), p * TSLOT + jnp.maximum(nt - 1, 0), jnp.int32)
        plsc.store_scatter(
            tval_v, [lane + (p * TSLOT + nt)], plsc.load_gather(tval_v, [last]))
        ntb = (nt + (TB - 1)) // TB

        def fire(j, fc):
            tid = tval_v[pl.ds(p * TSLOT + j * TB, TB)] >> 9
            pltpu.async_copy(
                trainable_hbm.at[tid],
                tbuf.at[pl.ds(p * TBH + j * TB, TB)], semt[p])
            return fc

        lax.fori_loop(0, ntb, fire, 0)
        return ntb

    def stage_finish(ci, p, ntb):
        for j in range(NBLK):
            pltpu.make_async_copy(
                frozen_hbm.at[pl.ds(0, GBLK)],
                obuf.at[pl.ds(0, GBLK)], semf[p]).wait()

        def drain(j, dc):
            pltpu.make_async_copy(
                trainable_hbm.at[pl.ds(0, TB)],
                tbuf.at[pl.ds(0, TB)], semt[p]).wait()
            return dc

        lax.fori_loop(0, ntb, drain, 0)

        def merge(j, mc):
            v = tval_v[pl.ds(p * TSLOT + j * TB, TB)]
            rows = (v & (CH - 1)) + p * CH
            lids = lane + (p * TBH + j * TB)
            for col in range(DIM):
                cs = jnp.full((LANES,), col, jnp.int32)
                vals = plsc.load_gather(tbuf, [lids, cs])
                plsc.store_scatter(obuf, [rows, cs], vals)
            return mc

        lax.fori_loop(0, ntb, merge, 0)
        pltpu.async_copy(
            obuf.at[pl.ds(p * CH, CH)],
            out_hbm.at[pl.ds(base + ci * CH, CH)], semo[p])

    def wait_store(p):
        pltpu.make_async_copy(
            obuf.at[pl.ds(0, CH)], out_hbm.at[pl.ds(0, CH)], semo[p]).wait()

    def iter_body(k, ntb_prev):
        @pl.when(k >= 1)
        def _w0():
            wait_store(0)

        ntb0 = stage_issue(2 * k, 0)

        @pl.when(k >= 1)
        def _f1():
            stage_finish(2 * k - 1, 1, ntb_prev)
            wait_store(1)

        ntb1 = stage_issue(2 * k + 1, 1)
        stage_finish(2 * k, 0, ntb0)
        return ntb1

    ntb_last = lax.fori_loop(0, n_chunks // 2, iter_body, jnp.int32(0))
    stage_finish(n_chunks - 1, 1, ntb_last)
    wait_store(0)
    wait_store(1)


def kernel(frozen_weight, trainable_buffer, input_ids):
    b, s = input_ids.shape
    n = b * s
    assert n % (NW * CH * 2) == 0
    n_chunks = n // (NW * CH)
    idx_flat = input_ids.reshape(n)

    k = pl.kernel(
        functools.partial(_body, n_chunks),
        out_type=jax.ShapeDtypeStruct((n, DIM), jnp.float32),
        mesh=plsc.VectorSubcoreMesh(core_axis_name="c", subcore_axis_name="s"),
        compiler_params=pltpu.CompilerParams(
            use_tc_tiling_on_sc=False, needs_layout_passes=False),
        scratch_types=[
            pltpu.VMEM((CH,), jnp.int32),
            pltpu.VMEM((2 * NBLK, GBLK), jnp.int32),
            pltpu.VMEM((2 * TSLOT,), jnp.int32),
            pltpu.VMEM((2 * TBH, DIM), jnp.float32),
            pltpu.VMEM((2 * CH, DIM), jnp.float32),
            pltpu.SemaphoreType.DMA,
            pltpu.SemaphoreType.DMA,
            pltpu.SemaphoreType.DMA,
            pltpu.SemaphoreType.DMA,
            pltpu.SemaphoreType.DMA,
            pltpu.SemaphoreType.DMA,
        ],
    )
    out = k(frozen_weight, trainable_buffer, idx_flat)
    return out.reshape(b, s, DIM)


# speedup vs baseline: 1.0325x; 1.0325x over previous
import functools

import jax
import jax.numpy as jnp
from jax import lax
from jax.experimental import pallas as pl
from jax.experimental.pallas import tpu as pltpu
from jax.experimental.pallas import tpu_sc as plsc

TRAIN_START = 900000
DIM = 32
LANES = 16

NC = 2
NS = 16
NW = NC * NS

CH = 512
GBLK = 128
NBLK = CH // GBLK
TB = LANES
TSLOT = CH + 2 * LANES
TBH = CH + LANES


def _body(n_chunks, frozen_hbm, trainable_hbm, idx_hbm, out_hbm,
          idx_v, fidx_v, tval_v, tbuf, obuf,
          semf0, semf1, semt0, semt1, semo0, semo1, semi):
    c = lax.axis_index("c")
    s = lax.axis_index("s")
    wid = s * NC + c
    base = wid * (n_chunks * CH)
    lane = lax.iota(jnp.int32, LANES)
    semf = (semf0, semf1)
    semt = (semt0, semt1)
    semo = (semo0, semo1)

    def stage_issue(ci, p):
        pltpu.make_async_copy(
            idx_hbm.at[pl.ds(0, CH)], idx_v.at[pl.ds(0, CH)], semi).wait()
        nci = jnp.minimum(ci + 1, n_chunks - 1)
        pltpu.async_copy(
            idx_hbm.at[pl.ds(base + nci * CH, CH)],
            idx_v.at[pl.ds((1 - p) * CH, CH)], semi)

        nt = jnp.int32(0)
        for g in range(CH // LANES):
            iv = idx_v[pl.ds(p * CH + g * LANES, LANES)]
            is_t = iv >= TRAIN_START
            dummy = lane + (wid * CH + g * LANES)
            fidx_v[p * NBLK + g // 8, pl.ds((g % 8) * LANES, LANES)] = (
                jnp.where(is_t, dummy, iv))
            packed = ((iv - TRAIN_START) << 9) | (lane + g * LANES)
            cnt = lax.cumsum(is_t.astype(jnp.int32), axis=0)
            pos = jnp.where(is_t, (p * TSLOT + nt - 1) + cnt,
                            (p * TSLOT + CH + LANES) + lane)
            plsc.store_scatter(tval_v, [pos], packed)
            nt = nt + cnt[LANES - 1]

        for j in range(NBLK):
            pltpu.async_copy(
                frozen_hbm.at[fidx_v.at[p * NBLK + j]],
                obuf.at[pl.ds(p * CH + j * GBLK, GBLK)], semf[p])

        last = jnp.full((LANES,), p * TSLOT + jnp.maximum(nt - 1, 0), jnp.int32)
        plsc.store_scatter(
            tval_v, [lane + (p * TSLOT + nt)], plsc.load_gather(tval_v, [last]))
        ntb = (nt + (TB - 1)) // TB

        def fire(j, fc):
            tid = tval_v[pl.ds(p * TSLOT + j * TB, TB)] >> 9
            pltpu.async_copy(
                trainable_hbm.at[tid],
                tbuf.at[pl.ds(p * TBH + j * TB, TB)], semt[p])
            return fc

        lax.fori_loop(0, ntb, fire, 0)
        return ntb

    def stage_finish(ci, p, ntb):
        for j in range(NBLK):
            pltpu.make_async_copy(
                frozen_hbm.at[pl.ds(0, GBLK)],
                obuf.at[pl.ds(0, GBLK)], semf[p]).wait()

        def drain(j, dc):
            pltpu.make_async_copy(
                trainable_hbm.at[pl.ds(0, TB)],
                tbuf.at[pl.ds(0, TB)], semt[p]).wait()
            return dc

        lax.fori_loop(0, ntb, drain, 0)

        def merge(j, mc):
            v = tval_v[pl.ds(p * TSLOT + j * TB, TB)]
            rows = (v & (CH - 1)) + p * CH
            lids = lane + (p * TBH + j * TB)
            for col in range(DIM):
                cs = jnp.full((LANES,), col, jnp.int32)
                vals = plsc.load_gather(tbuf, [lids, cs])
                plsc.store_scatter(obuf, [rows, cs], vals)
            return mc

        lax.fori_loop(0, ntb, merge, 0)
        pltpu.async_copy(
            obuf.at[pl.ds(p * CH, CH)],
            out_hbm.at[pl.ds(base + ci * CH, CH)], semo[p])

    def wait_store(p):
        pltpu.make_async_copy(
            obuf.at[pl.ds(0, CH)], out_hbm.at[pl.ds(0, CH)], semo[p]).wait()

    def iter_body(k, ntb_prev):
        @pl.when(k >= 1)
        def _w0():
            wait_store(0)

        ntb0 = stage_issue(2 * k, 0)

        @pl.when(k >= 1)
        def _f1():
            stage_finish(2 * k - 1, 1, ntb_prev)
            wait_store(1)

        ntb1 = stage_issue(2 * k + 1, 1)
        stage_finish(2 * k, 0, ntb0)
        return ntb1

    pltpu.async_copy(
        idx_hbm.at[pl.ds(base, CH)], idx_v.at[pl.ds(0, CH)], semi)
    ntb_last = lax.fori_loop(0, n_chunks // 2, iter_body, jnp.int32(0))
    stage_finish(n_chunks - 1, 1, ntb_last)
    pltpu.make_async_copy(
        idx_hbm.at[pl.ds(0, CH)], idx_v.at[pl.ds(0, CH)], semi).wait()
    wait_store(0)
    wait_store(1)


def kernel(frozen_weight, trainable_buffer, input_ids):
    b, s = input_ids.shape
    n = b * s
    assert n % (NW * CH * 2) == 0
    n_chunks = n // (NW * CH)
    idx_flat = input_ids.reshape(n)

    k = pl.kernel(
        functools.partial(_body, n_chunks),
        out_type=jax.ShapeDtypeStruct((n, DIM), jnp.float32),
        mesh=plsc.VectorSubcoreMesh(core_axis_name="c", subcore_axis_name="s"),
        compiler_params=pltpu.CompilerParams(
            use_tc_tiling_on_sc=False, needs_layout_passes=False),
        scratch_types=[
            pltpu.VMEM((2 * CH,), jnp.int32),
            pltpu.VMEM((2 * NBLK, GBLK), jnp.int32),
            pltpu.VMEM((2 * TSLOT,), jnp.int32),
            pltpu.VMEM((2 * TBH, DIM), jnp.float32),
            pltpu.VMEM((2 * CH, DIM), jnp.float32),
            pltpu.SemaphoreType.DMA,
            pltpu.SemaphoreType.DMA,
            pltpu.SemaphoreType.DMA,
            pltpu.SemaphoreType.DMA,
            pltpu.SemaphoreType.DMA,
            pltpu.SemaphoreType.DMA,
            pltpu.SemaphoreType.DMA,
        ],
    )
    out = k(frozen_weight, trainable_buffer, idx_flat)
    return out.reshape(b, s, DIM)


# speedup vs baseline: 1.0382x; 1.0055x over previous
import functools

import jax
import jax.numpy as jnp
from jax import lax
from jax.experimental import pallas as pl
from jax.experimental.pallas import tpu as pltpu
from jax.experimental.pallas import tpu_sc as plsc

TRAIN_START = 900000
DIM = 32
LANES = 16

NC = 2
NS = 16
NW = NC * NS

CH = 640
GBLK = 128
NBLK = CH // GBLK
TB = LANES
RBITS = (CH - 1).bit_length()
RMASK = (1 << RBITS) - 1
TSLOT = CH + 2 * LANES
TBH = CH + LANES


def _body(n_chunks, frozen_hbm, trainable_hbm, idx_hbm, out_hbm,
          idx_v, fidx_v, tval_v, tbuf, obuf,
          semf0, semf1, semt0, semt1, semo0, semo1, semi):
    c = lax.axis_index("c")
    s = lax.axis_index("s")
    wid = s * NC + c
    base = wid * (n_chunks * CH)
    lane = lax.iota(jnp.int32, LANES)
    semf = (semf0, semf1)
    semt = (semt0, semt1)
    semo = (semo0, semo1)

    def stage_issue(ci, p):
        pltpu.make_async_copy(
            idx_hbm.at[pl.ds(0, CH)], idx_v.at[pl.ds(0, CH)], semi).wait()
        nci = jnp.minimum(ci + 1, n_chunks - 1)
        pltpu.async_copy(
            idx_hbm.at[pl.ds(base + nci * CH, CH)],
            idx_v.at[pl.ds((1 - p) * CH, CH)], semi)

        nt = jnp.int32(0)
        for g in range(CH // LANES):
            iv = idx_v[pl.ds(p * CH + g * LANES, LANES)]
            is_t = iv >= TRAIN_START
            dummy = lane + (wid * CH + g * LANES)
            fidx_v[p * NBLK + g // 8, pl.ds((g % 8) * LANES, LANES)] = (
                jnp.where(is_t, dummy, iv))
            packed = ((iv - TRAIN_START) << RBITS) | (lane + g * LANES)
            cnt = lax.cumsum(is_t.astype(jnp.int32), axis=0)
            pos = jnp.where(is_t, (p * TSLOT + nt - 1) + cnt,
                            (p * TSLOT + CH + LANES) + lane)
            plsc.store_scatter(tval_v, [pos], packed)
            nt = nt + cnt[LANES - 1]

        for j in range(NBLK):
            pltpu.async_copy(
                frozen_hbm.at[fidx_v.at[p * NBLK + j]],
                obuf.at[pl.ds(p * CH + j * GBLK, GBLK)], semf[p])

        last = jnp.full((LANES,), p * TSLOT + jnp.maximum(nt - 1, 0), jnp.int32)
        plsc.store_scatter(
            tval_v, [lane + (p * TSLOT + nt)], plsc.load_gather(tval_v, [last]))
        ntb = (nt + (TB - 1)) // TB

        def fire(j, fc):
            tid = tval_v[pl.ds(p * TSLOT + j * TB, TB)] >> RBITS
            pltpu.async_copy(
                trainable_hbm.at[tid],
                tbuf.at[pl.ds(p * TBH + j * TB, TB)], semt[p])
            return fc

        lax.fori_loop(0, ntb, fire, 0)
        return ntb

    def stage_finish(ci, p, ntb):
        for j in range(NBLK):
            pltpu.make_async_copy(
                frozen_hbm.at[pl.ds(0, GBLK)],
                obuf.at[pl.ds(0, GBLK)], semf[p]).wait()

        def drain(j, dc):
            pltpu.make_async_copy(
                trainable_hbm.at[pl.ds(0, TB)],
                tbuf.at[pl.ds(0, TB)], semt[p]).wait()
            return dc

        lax.fori_loop(0, ntb, drain, 0)

        def merge(j, mc):
            v = tval_v[pl.ds(p * TSLOT + j * TB, TB)]
            rows = (v & RMASK) + p * CH
            lids = lane + (p * TBH + j * TB)
            for col in range(DIM):
                cs = jnp.full((LANES,), col, jnp.int32)
                vals = plsc.load_gather(tbuf, [lids, cs])
                plsc.store_scatter(obuf, [rows, cs], vals)
            return mc

        lax.fori_loop(0, ntb, merge, 0)
        pltpu.async_copy(
            obuf.at[pl.ds(p * CH, CH)],
            out_hbm.at[pl.ds(base + ci * CH, CH)], semo[p])

    def wait_store(p):
        pltpu.make_async_copy(
            obuf.at[pl.ds(0, CH)], out_hbm.at[pl.ds(0, CH)], semo[p]).wait()

    def iter_body(k, ntb_prev):
        @pl.when(k >= 1)
        def _w0():
            wait_store(0)

        ntb0 = stage_issue(2 * k, 0)

        @pl.when(k >= 1)
        def _f1():
            stage_finish(2 * k - 1, 1, ntb_prev)
            wait_store(1)

        ntb1 = stage_issue(2 * k + 1, 1)
        stage_finish(2 * k, 0, ntb0)
        return ntb1

    pltpu.async_copy(
        idx_hbm.at[pl.ds(base, CH)], idx_v.at[pl.ds(0, CH)], semi)
    ntb_last = lax.fori_loop(0, n_chunks // 2, iter_body, jnp.int32(0))
    stage_finish(n_chunks - 1, 1, ntb_last)
    pltpu.make_async_copy(
        idx_hbm.at[pl.ds(0, CH)], idx_v.at[pl.ds(0, CH)], semi).wait()
    wait_store(0)
    wait_store(1)


def kernel(frozen_weight, trainable_buffer, input_ids):
    b, s = input_ids.shape
    n = b * s
    assert n % (NW * CH * 2) == 0
    n_chunks = n // (NW * CH)
    idx_flat = input_ids.reshape(n)

    k = pl.kernel(
        functools.partial(_body, n_chunks),
        out_type=jax.ShapeDtypeStruct((n, DIM), jnp.float32),
        mesh=plsc.VectorSubcoreMesh(core_axis_name="c", subcore_axis_name="s"),
        compiler_params=pltpu.CompilerParams(
            use_tc_tiling_on_sc=False, needs_layout_passes=False),
        scratch_types=[
            pltpu.VMEM((2 * CH,), jnp.int32),
            pltpu.VMEM((2 * NBLK, GBLK), jnp.int32),
            pltpu.VMEM((2 * TSLOT,), jnp.int32),
            pltpu.VMEM((2 * TBH, DIM), jnp.float32),
            pltpu.VMEM((2 * CH, DIM), jnp.float32),
            pltpu.SemaphoreType.DMA,
            pltpu.SemaphoreType.DMA,
            pltpu.SemaphoreType.DMA,
            pltpu.SemaphoreType.DMA,
            pltpu.SemaphoreType.DMA,
            pltpu.SemaphoreType.DMA,
            pltpu.SemaphoreType.DMA,
        ],
    )
    out = k(frozen_weight, trainable_buffer, idx_flat)
    return out.reshape(b, s, DIM)
